# Initial kernel scaffold; baseline (speedup 1.0000x reference)
import functools

import jax
import jax.numpy as jnp
from jax import lax
from jax.experimental import pallas as pl
from jax.experimental.pallas import tpu as pltpu
from jax.experimental.pallas import tpu_sc as plsc

B = 16384
H = 50
D = 64
C = 20
LANES = 16
NC, NS = 2, 16
NW = NC * NS
BAGS_PER_W = B // NW
GPB = 2
IDXS = GPB * H
STEPS = BAGS_PER_W // GPB
NBUF = 4
INV_H = 1.0 / H


def _sc_embedding_bag(text2, emb_weight):
    mesh = plsc.VectorSubcoreMesh(core_axis_name="c", subcore_axis_name="s")

    @functools.partial(
        pl.kernel,
        out_type=jax.ShapeDtypeStruct((B, D), jnp.float32),
        mesh=mesh,
        scratch_types=[
            pltpu.VMEM((STEPS, IDXS), jnp.int32),
            pltpu.VMEM((NBUF, IDXS, D), jnp.float32),
            pltpu.VMEM((BAGS_PER_W, D), jnp.float32),
        ] + [pltpu.SemaphoreType.DMA] * NBUF,
    )
    def k(text_hbm, emb_hbm, out_hbm, idx_v, rows_v, pooled_v,
          sem0, sem1, sem2, sem3):
        sems = (sem0, sem1, sem2, sem3)
        wid = lax.axis_index("c") * NS + lax.axis_index("s")

        pltpu.sync_copy(text_hbm.at[wid], idx_v)

        def start(g, b):
            pltpu.async_copy(emb_hbm.at[idx_v.at[g]], rows_v.at[b], sems[b])

        def compute(g, b):
            pltpu.make_async_copy(
                emb_hbm.at[idx_v.at[g]], rows_v.at[b], sems[b]).wait()
            for bag in range(GPB):
                def jbody(j, accs):
                    r = bag * H + j
                    return tuple(
                        accs[c] + rows_v[b, r, pl.ds(c * LANES, LANES)]
                        for c in range(D // LANES))
                accs = lax.fori_loop(
                    0, H, jbody,
                    tuple(jnp.zeros((LANES,), jnp.float32)
                          for _ in range(D // LANES)),
                    unroll=2)
                row = g * GPB + bag
                for c in range(D // LANES):
                    pooled_v[row, pl.ds(c * LANES, LANES)] = accs[c] * INV_H

        for b in range(NBUF):
            start(b, b)

        def obody(i, carry):
            g0 = i * NBUF
            for b in range(NBUF):
                compute(g0 + b, b)
                start(g0 + b + NBUF, b)
            return carry

        lax.fori_loop(0, STEPS // NBUF - 1, obody, 0)

        for b in range(NBUF):
            compute(STEPS - NBUF + b, b)

        pltpu.sync_copy(pooled_v, out_hbm.at[pl.ds(wid * BAGS_PER_W,
                                                   BAGS_PER_W), :])

    return k(text2, emb_weight)


def _tc_linear(pooled, fc_w, fc_b2):
    BM = 2048

    def mm(x_ref, w_ref, b_ref, o_ref):
        o_ref[...] = lax.dot_general(
            x_ref[...], w_ref[...], (((1,), (1,)), ((), ())),
            preferred_element_type=jnp.float32) + b_ref[...]

    return pl.pallas_call(
        mm,
        grid=(B // BM,),
        in_specs=[
            pl.BlockSpec((BM, D), lambda i: (i, 0)),
            pl.BlockSpec((C, D), lambda i: (0, 0)),
            pl.BlockSpec((1, C), lambda i: (0, 0)),
        ],
        out_specs=pl.BlockSpec((BM, C), lambda i: (i, 0)),
        out_shape=jax.ShapeDtypeStruct((B, C), jnp.float32),
    )(pooled, fc_w, fc_b2)


@jax.jit
def kernel(text, emb_weight, fc_w, fc_b):
    text2 = text.astype(jnp.int32).reshape(NW, STEPS, IDXS)
    pooled = _sc_embedding_bag(text2, emb_weight)
    return _tc_linear(pooled, fc_w, fc_b.reshape(1, C))


# speedup vs baseline: 2.7778x; 2.7778x over previous
import functools

import jax
import jax.numpy as jnp
from jax import lax
from jax.experimental import pallas as pl
from jax.experimental.pallas import tpu as pltpu
from jax.experimental.pallas import tpu_sc as plsc

B = 16384
H = 50
D = 64
C = 20
LANES = 16
NC, NS = 2, 16
NW = NC * NS
BAGS_PER_W = B // NW
GPB = 2
IDXS = GPB * H
STEPS = BAGS_PER_W // GPB
NBUF = 4
INV_H = 1.0 / H


def _sc_embedding_bag(text2, emb_weight):
    mesh = plsc.VectorSubcoreMesh(core_axis_name="c", subcore_axis_name="s")

    @functools.partial(
        pl.kernel,
        out_type=jax.ShapeDtypeStruct((B, D), jnp.float32),
        mesh=mesh,
        scratch_types=[
            pltpu.VMEM((STEPS, IDXS), jnp.int32),
            pltpu.VMEM((NBUF, IDXS, D), jnp.float32),
            pltpu.VMEM((BAGS_PER_W, D), jnp.float32),
        ] + [pltpu.SemaphoreType.DMA] * NBUF,
        compiler_params=pltpu.CompilerParams(use_tc_tiling_on_sc=False),
    )
    def k(text_hbm, emb_hbm, out_hbm, idx_v, rows_v, pooled_v,
          sem0, sem1, sem2, sem3):
        sems = (sem0, sem1, sem2, sem3)
        wid = lax.axis_index("c") * NS + lax.axis_index("s")

        pltpu.sync_copy(text_hbm.at[wid], idx_v)

        def start(g, b):
            pltpu.async_copy(emb_hbm.at[idx_v.at[g]], rows_v.at[b], sems[b])

        def compute(g, b):
            pltpu.make_async_copy(
                emb_hbm.at[idx_v.at[g]], rows_v.at[b], sems[b]).wait()
            for bag in range(GPB):
                def jbody(j, accs):
                    r = bag * H + j
                    return tuple(
                        accs[c] + rows_v[b, r, pl.ds(c * LANES, LANES)]
                        for c in range(D // LANES))
                accs = lax.fori_loop(
                    0, H, jbody,
                    tuple(jnp.zeros((LANES,), jnp.float32)
                          for _ in range(D // LANES)),
                    unroll=2)
                row = g * GPB + bag
                for c in range(D // LANES):
                    pooled_v[row, pl.ds(c * LANES, LANES)] = accs[c] * INV_H

        for b in range(NBUF):
            start(b, b)

        def obody(i, carry):
            g0 = i * NBUF
            for b in range(NBUF):
                compute(g0 + b, b)
                start(g0 + b + NBUF, b)
            return carry

        lax.fori_loop(0, STEPS // NBUF - 1, obody, 0)

        for b in range(NBUF):
            compute(STEPS - NBUF + b, b)

        pltpu.sync_copy(pooled_v, out_hbm.at[pl.ds(wid * BAGS_PER_W,
                                                   BAGS_PER_W), :])

    return k(text2, emb_weight)


def _tc_linear(pooled, fc_w, fc_b2):
    BM = 2048

    def mm(x_ref, w_ref, b_ref, o_ref):
        o_ref[...] = lax.dot_general(
            x_ref[...], w_ref[...], (((1,), (1,)), ((), ())),
            preferred_element_type=jnp.float32) + b_ref[...]

    return pl.pallas_call(
        mm,
        grid=(B // BM,),
        in_specs=[
            pl.BlockSpec((BM, D), lambda i: (i, 0)),
            pl.BlockSpec((C, D), lambda i: (0, 0)),
            pl.BlockSpec((1, C), lambda i: (0, 0)),
        ],
        out_specs=pl.BlockSpec((BM, C), lambda i: (i, 0)),
        out_shape=jax.ShapeDtypeStruct((B, C), jnp.float32),
    )(pooled, fc_w, fc_b2)


@jax.jit
def kernel(text, emb_weight, fc_w, fc_b):
    text2 = text.astype(jnp.int32).reshape(NW, STEPS, IDXS)
    pooled = _sc_embedding_bag(text2, emb_weight)
    return _tc_linear(pooled, fc_w, fc_b.reshape(1, C))


# speedup vs baseline: 2.8297x; 1.0187x over previous
import functools

import jax
import jax.numpy as jnp
from jax import lax
from jax.experimental import pallas as pl
from jax.experimental.pallas import tpu as pltpu
from jax.experimental.pallas import tpu_sc as plsc

B = 16384
H = 50
D = 64
C = 20
LANES = 16
NC, NS = 2, 16
NW = NC * NS
BAGS_PER_W = B // NW
GPB = 2
IDXS = GPB * H
STEPS = BAGS_PER_W // GPB
NBUF = 8
INV_H = 1.0 / H


def _sc_embedding_bag(text2, emb_weight):
    mesh = plsc.VectorSubcoreMesh(core_axis_name="c", subcore_axis_name="s")

    @functools.partial(
        pl.kernel,
        out_type=jax.ShapeDtypeStruct((B, D), jnp.float32),
        mesh=mesh,
        scratch_types=[
            pltpu.VMEM((STEPS, IDXS), jnp.int32),
            pltpu.VMEM((NBUF, IDXS, D), jnp.float32),
            pltpu.VMEM((BAGS_PER_W, D), jnp.float32),
        ] + [pltpu.SemaphoreType.DMA] * NBUF,
        compiler_params=pltpu.CompilerParams(use_tc_tiling_on_sc=False),
    )
    def k(text_hbm, emb_hbm, out_hbm, idx_v, rows_v, pooled_v,
          *sems):
        wid = lax.axis_index("c") * NS + lax.axis_index("s")

        pltpu.sync_copy(text_hbm.at[wid], idx_v)

        def start(g, b):
            pltpu.async_copy(emb_hbm.at[idx_v.at[g]], rows_v.at[b], sems[b])

        def compute(g, b):
            pltpu.make_async_copy(
                emb_hbm.at[idx_v.at[g]], rows_v.at[b], sems[b]).wait()
            for bag in range(GPB):
                def jbody(j, accs):
                    r = bag * H + j
                    return tuple(
                        accs[c] + rows_v[b, r, pl.ds(c * LANES, LANES)]
                        for c in range(D // LANES))
                accs = lax.fori_loop(
                    0, H, jbody,
                    tuple(jnp.zeros((LANES,), jnp.float32)
                          for _ in range(D // LANES)),
                    unroll=2)
                row = g * GPB + bag
                for c in range(D // LANES):
                    pooled_v[row, pl.ds(c * LANES, LANES)] = accs[c] * INV_H

        for b in range(NBUF):
            start(b, b)

        def obody(i, carry):
            g0 = i * NBUF
            for b in range(NBUF):
                compute(g0 + b, b)
                start(g0 + b + NBUF, b)
            return carry

        lax.fori_loop(0, STEPS // NBUF - 1, obody, 0)

        for b in range(NBUF):
            compute(STEPS - NBUF + b, b)

        pltpu.sync_copy(pooled_v, out_hbm.at[pl.ds(wid * BAGS_PER_W,
                                                   BAGS_PER_W), :])

    return k(text2, emb_weight)


def _tc_linear(pooled, fc_w, fc_b2):
    BM = 2048

    def mm(x_ref, w_ref, b_ref, o_ref):
        o_ref[...] = lax.dot_general(
            x_ref[...], w_ref[...], (((1,), (1,)), ((), ())),
            preferred_element_type=jnp.float32) + b_ref[...]

    return pl.pallas_call(
        mm,
        grid=(B // BM,),
        in_specs=[
            pl.BlockSpec((BM, D), lambda i: (i, 0)),
            pl.BlockSpec((C, D), lambda i: (0, 0)),
            pl.BlockSpec((1, C), lambda i: (0, 0)),
        ],
        out_specs=pl.BlockSpec((BM, C), lambda i: (i, 0)),
        out_shape=jax.ShapeDtypeStruct((B, C), jnp.float32),
    )(pooled, fc_w, fc_b2)


@jax.jit
def kernel(text, emb_weight, fc_w, fc_b):
    text2 = text.astype(jnp.int32).reshape(NW, STEPS, IDXS)
    pooled = _sc_embedding_bag(text2, emb_weight)
    return _tc_linear(pooled, fc_w, fc_b.reshape(1, C))
